# baseline (speedup 1.0000x reference)
import functools

import jax
import jax.numpy as jnp
from jax import lax
from jax.experimental import pallas as pl
from jax.experimental.pallas import tpu as pltpu
from jax.experimental.pallas import tpu_sc as plsc

_D = 64
_T = 200
_B = 4096
_SCALE = 8.0

_info = plsc.get_sparse_core_info()
_NC, _NS, _L = _info.num_cores, _info.num_subcores, _info.num_lanes
_NW = _NC * _NS
_SEQ_PER_W = _B // _NW
_CHUNK_A = 128
_CHUNK_B = _T - _CHUNK_A


@functools.partial(
    pl.kernel,
    mesh=plsc.VectorSubcoreMesh(core_axis_name="c", subcore_axis_name="s"),
    compiler_params=pltpu.CompilerParams(use_tc_tiling_on_sc=False),
    out_type=jax.ShapeDtypeStruct((_B * _T, _D), jnp.float32),
    scratch_types=[
        pltpu.VMEM((_T, _D), jnp.float32),
        pltpu.VMEM((_CHUNK_A,), jnp.int32),
        pltpu.VMEM((_CHUNK_B,), jnp.int32),
        pltpu.VMEM((_T, _D), jnp.float32),
        pltpu.SemaphoreType.DMA,
    ],
)
def _embed(x_hbm, tok_hbm, pos_hbm, out_hbm, pos_v, idx_a, idx_b, rows_v, sem):
    wid = lax.axis_index("s") * _NC + lax.axis_index("c")
    pltpu.sync_copy(pos_hbm.at[pl.ds(0, _T)], pos_v)

    def seq_body(s, carry):
        base = (wid * _SEQ_PER_W + s) * _T
        pltpu.sync_copy(x_hbm.at[pl.ds(base, _CHUNK_A)], idx_a)
        pltpu.sync_copy(x_hbm.at[pl.ds(base + _CHUNK_A, _CHUNK_B)], idx_b)
        cp1 = pltpu.async_copy(tok_hbm.at[idx_a], rows_v.at[pl.ds(0, _CHUNK_A)], sem)
        cp2 = pltpu.async_copy(tok_hbm.at[idx_b], rows_v.at[pl.ds(_CHUNK_A, _CHUNK_B)], sem)
        cp1.wait()
        cp2.wait()

        def row_body(r, c2):
            for j in range(_D // _L):
                sl = pl.ds(j * _L, _L)
                rows_v[r, sl] = rows_v[r, sl] * _SCALE + pos_v[r, sl]
            return c2

        lax.fori_loop(0, _T, row_body, 0)
        pltpu.sync_copy(rows_v, out_hbm.at[pl.ds(base, _T)])
        return carry

    lax.fori_loop(0, _SEQ_PER_W, seq_body, 0)


def kernel(x, token_table, pos_table):
    b, t = x.shape
    out = _embed(x.reshape(b * t).astype(jnp.int32), token_table, pos_table)
    return out.reshape(b, t, _D)


# speedup vs baseline: 1.2544x; 1.2544x over previous
import functools

import jax
import jax.numpy as jnp
from jax import lax
from jax.experimental import pallas as pl
from jax.experimental.pallas import tpu as pltpu
from jax.experimental.pallas import tpu_sc as plsc

_D = 64
_T = 200
_B = 4096
_SCALE = 8.0

_info = plsc.get_sparse_core_info()
_NC, _NS, _L = _info.num_cores, _info.num_subcores, _info.num_lanes
_NW = _NC * _NS
_SEQ_PER_W = _B // _NW
_CHUNK_A = 128
_CHUNK_B = _T - _CHUNK_A
_NBUF = 2


@functools.partial(
    pl.kernel,
    mesh=plsc.VectorSubcoreMesh(core_axis_name="c", subcore_axis_name="s"),
    compiler_params=pltpu.CompilerParams(use_tc_tiling_on_sc=False),
    out_type=jax.ShapeDtypeStruct((_B * _T, _D), jnp.float32),
    scratch_types=[
        pltpu.VMEM((_T, _D), jnp.float32),
        pltpu.VMEM((_SEQ_PER_W * _T,), jnp.int32),
        pltpu.VMEM((_T, _D), jnp.float32),
        pltpu.VMEM((_T, _D), jnp.float32),
        pltpu.VMEM((_T, _D), jnp.float32),
        pltpu.VMEM((_T, _D), jnp.float32),
        pltpu.SemaphoreType.DMA,
        pltpu.SemaphoreType.DMA,
        pltpu.SemaphoreType.DMA,
        pltpu.SemaphoreType.DMA,
    ],
)
def _embed(x_hbm, tok_hbm, pos_hbm, out_hbm,
           pos_v, idx_v, rin0, rin1, rout0, rout1, gs0, gs1, ss0, ss1):
    rin = (rin0, rin1)
    rout = (rout0, rout1)
    gsem = (gs0, gs1)
    ssem = (ss0, ss1)
    wid = lax.axis_index("s") * _NC + lax.axis_index("c")
    seq0 = wid * _SEQ_PER_W

    pltpu.sync_copy(pos_hbm.at[pl.ds(0, _T)], pos_v)
    pltpu.sync_copy(x_hbm.at[pl.ds(seq0 * _T, _SEQ_PER_W * _T)], idx_v)

    def issue_gather(b, s):
        off = s * _T
        pltpu.async_copy(
            tok_hbm.at[idx_v.at[pl.ds(off, _CHUNK_A)]],
            rin[b].at[pl.ds(0, _CHUNK_A)], gsem[b])
        pltpu.async_copy(
            tok_hbm.at[idx_v.at[pl.ds(off + _CHUNK_A, _CHUNK_B)]],
            rin[b].at[pl.ds(_CHUNK_A, _CHUNK_B)], gsem[b])

    def wait_gather(b):
        pltpu.make_async_copy(tok_hbm.at[pl.ds(0, _T)], rin[b], gsem[b]).wait()

    def issue_store(b, s):
        pltpu.async_copy(rout[b], out_hbm.at[pl.ds((seq0 + s) * _T, _T)], ssem[b])

    def wait_store(b):
        pltpu.make_async_copy(rout[b], out_hbm.at[pl.ds(0, _T)], ssem[b]).wait()

    for b in range(_NBUF):
        issue_gather(b, b)

    def round_body(k, carry):
        for b in range(_NBUF):
            s = k * _NBUF + b
            wait_gather(b)

            @pl.when(k > 0)
            def _():
                wait_store(b)

            rin_b, rout_b = rin[b], rout[b]

            @plsc.parallel_loop(0, _T, unroll=8)
            def _(r):
                for j in range(_D // _L):
                    sl = pl.ds(j * _L, _L)
                    rout_b[r, sl] = rin_b[r, sl] * _SCALE + pos_v[r, sl]

            @pl.when(s + _NBUF < _SEQ_PER_W)
            def _():
                issue_gather(b, s + _NBUF)

            issue_store(b, s)
        return carry

    lax.fori_loop(0, _SEQ_PER_W // _NBUF, round_body, 0)
    for b in range(_NBUF):
        wait_store(b)


def kernel(x, token_table, pos_table):
    b, t = x.shape
    out = _embed(x.reshape(b * t).astype(jnp.int32), token_table, pos_table)
    return out.reshape(b, t, _D)
